# baseline (speedup 1.0000x reference)
import functools

import jax
import jax.numpy as jnp
from jax.experimental import pallas as pl
from jax.experimental.pallas import tpu as pltpu
from jax.experimental.pallas import tpu_sc as plsc

_WINDOW = 128


def _sc_gather_all(idx_w, idx_c, idx_p, wt, ct, pt):
    wd = wt.shape[1]
    cd = ct.shape[1]
    pd = pt.shape[1]
    n_tok = idx_w.shape[1]
    n_char = idx_c.shape[1]
    mesh = plsc.VectorSubcoreMesh(core_axis_name="c", subcore_axis_name="s")
    out_types = (
        jax.ShapeDtypeStruct((n_tok, wd), wt.dtype),
        jax.ShapeDtypeStruct((n_char, cd), ct.dtype),
        jax.ShapeDtypeStruct((n_tok, pd), pt.dtype),
    )

    @functools.partial(pl.kernel, out_type=out_types, mesh=mesh)
    def k(wt_hbm, ct_hbm, pt_hbm, iw_hbm, ic_hbm, ip_hbm, ow_hbm, oc_hbm, op_hbm):
        def run(table_hbm, i_hbm, o_hbm, d):
            def body(i_vmem, o_vmem):
                pltpu.sync_copy(table_hbm.at[i_vmem.at[0]], o_vmem)

            pltpu.emit_pipeline(
                body,
                grid=(i_hbm.shape[1] // _WINDOW,),
                in_specs=[pl.BlockSpec((1, _WINDOW), lambda i: (0, i))],
                out_specs=[pl.BlockSpec((_WINDOW, d), lambda i: (i, 0))],
                core_axis_name=("c", "s"),
                dimension_semantics=(pltpu.PARALLEL,),
            )(i_hbm, o_hbm)

        run(wt_hbm, iw_hbm, ow_hbm, wd)
        run(ct_hbm, ic_hbm, oc_hbm, cd)
        run(pt_hbm, ip_hbm, op_hbm, pd)

    return k(wt, ct, pt, idx_w, idx_c, idx_p)


def _tc_combine(char_g, word_g, pos_g, wcat, bias2d, t_blk, tp, to, kk, pd):
    bl, wd = word_g.shape
    pdp = pos_g.shape[1]
    cdp = char_g.shape[1]
    cd = wcat.shape[0] // kk
    f = wcat.shape[1]

    def body(cg, wg, pg, wc, b, out):
        e = cg[...][:, 0:cd].astype(jnp.bfloat16)
        shifted = [e] + [
            jnp.concatenate([e[k:, :], e[:k, :]], axis=0) for k in range(1, kk)
        ]
        x = jnp.concatenate(shifted, axis=1)
        conv = jax.lax.dot_general(
            x, wc[...], (((1,), (0,)), ((), ())),
            preferred_element_type=jnp.float32,
        )
        conv = conv + b[...]
        conv = conv.reshape(t_blk, tp, f)
        t_idx = jax.lax.broadcasted_iota(jnp.int32, (t_blk, tp, f), 1)
        conv = jnp.where(t_idx < to, conv, -jnp.inf)
        feat = jnp.tanh(jnp.max(conv, axis=1))
        out[:, 0:wd] = wg[...]
        out[:, wd:wd + f] = feat
        out[:, wd + f:] = pg[...][:, 0:pd]

    return pl.pallas_call(
        body,
        grid=(bl // t_blk,),
        in_specs=[
            pl.BlockSpec((t_blk * tp, cdp), lambda i: (i, 0)),
            pl.BlockSpec((t_blk, wd), lambda i: (i, 0)),
            pl.BlockSpec((t_blk, pdp), lambda i: (i, 0)),
            pl.BlockSpec((kk * cd, f), lambda i: (0, 0)),
            pl.BlockSpec((1, f), lambda i: (0, 0)),
        ],
        out_specs=pl.BlockSpec((t_blk, wd + f + pd), lambda i: (i, 0)),
        out_shape=jax.ShapeDtypeStruct((bl, wd + f + pd), jnp.float32),
    )(char_g, word_g, pos_g, wcat, bias2d)


def kernel(input_word, input_char, input_pos, word_table, char_table, pos_table,
           conv_w, conv_b):
    b, l = input_word.shape
    lc = input_char.shape[2]
    f, cd, kk = conv_w.shape
    wd = word_table.shape[1]
    pd = pos_table.shape[1]
    bl = b * l
    tp = lc + 2 * (kk - 1)
    to = tp - kk + 1
    n_char_rows = char_table.shape[0]

    lane = 128
    ct = jnp.zeros((n_char_rows + 1, lane), char_table.dtype)
    ct = ct.at[:n_char_rows, :cd].set(char_table)
    pt = jnp.zeros((pos_table.shape[0], lane), pos_table.dtype)
    pt = pt.at[:, :pd].set(pos_table)
    idx_c = jnp.pad(
        input_char.reshape(bl, lc), ((0, 0), (kk - 1, kk - 1)),
        constant_values=n_char_rows,
    ).reshape(1, bl * tp).astype(jnp.int32)
    idx_w = input_word.reshape(1, bl).astype(jnp.int32)
    idx_p = input_pos.reshape(1, bl).astype(jnp.int32)

    word_g, char_g, pos_g = _sc_gather_all(
        idx_w, idx_c, idx_p, word_table, ct, pt)

    wcat = jnp.transpose(conv_w, (2, 1, 0)).reshape(kk * cd, f)
    wcat = wcat.astype(jnp.bfloat16)

    out = _tc_combine(char_g, word_g, pos_g, wcat, conv_b.reshape(1, f),
                      256, tp, to, kk, pd)
    return out.reshape(b, l, wd + f + pd)


# speedup vs baseline: 3.6365x; 3.6365x over previous
import dataclasses
import functools

import jax
import jax.numpy as jnp
from jax import lax
from jax.experimental import pallas as pl
from jax.experimental.pallas import tpu as pltpu
from jax.experimental.pallas import tpu_sc as plsc

_WINDOW = 128
_NC, _NS = 2, 16
_NW = _NC * _NS


def _vmem_gather_chunk(tab_v, idx_v, row_buf, n_rows, lanes):
    @pl.loop(0, n_rows // 16)
    def _(j):
        iv = idx_v[pl.ds(j * 16, 16)]
        src_base = iv * lanes
        dst_base = (j * 16 + lax.iota(jnp.int32, 16)) * lanes
        for c in range(lanes):
            v = plsc.load_gather(tab_v, [src_base + c])
            plsc.store_scatter(row_buf, [dst_base + c], v)


def _sc_gather_all(idx_w, idx_c, idx_p, wt, ctp, ptp):
    wd = wt.shape[1]
    ncr, cl = ctp.shape
    npr, plp = ptp.shape
    n_tok = idx_w.shape[1]
    n_char = idx_c.shape[0]
    c_per_w = n_char // _NW
    p_per_w = n_tok // _NW
    c_chunk = 512
    p_chunk = 400
    n_cch = c_per_w // c_chunk
    n_pch = p_per_w // p_chunk
    mesh = plsc.VectorSubcoreMesh(core_axis_name="c", subcore_axis_name="s")
    out_types = (
        jax.ShapeDtypeStruct((n_tok, wd), wt.dtype),
        jax.ShapeDtypeStruct((n_char * cl,), jnp.int32),
        jax.ShapeDtypeStruct((n_tok * plp,), jnp.int32),
    )
    scratch = [
        pltpu.VMEM((ncr * cl,), jnp.int32),
        pltpu.VMEM((npr * plp,), jnp.int32),
        pltpu.VMEM((c_chunk,), jnp.int32),
        pltpu.VMEM((c_chunk * cl,), jnp.int32),
        pltpu.VMEM((c_chunk * cl,), jnp.int32),
        pltpu.SemaphoreType.DMA,
        pltpu.SemaphoreType.DMA,
    ]

    cp = pltpu.CompilerParams()
    if "needs_layout_passes" in pltpu.CompilerParams.__dataclass_fields__:
        cp = dataclasses.replace(cp, needs_layout_passes=False)

    @functools.partial(pl.kernel, out_type=out_types, mesh=mesh,
                       scratch_types=scratch, compiler_params=cp)
    def k(wt_hbm, ctp_hbm, ptp_hbm, iw_hbm, ic_hbm, ip_hbm,
          ow_hbm, oc_hbm, op_hbm, tab_v, ptab_v, idx_v, rb0, rb1, s0, s1):
        wid = lax.axis_index("s") * _NC + lax.axis_index("c")
        pltpu.sync_copy(ctp_hbm, tab_v)
        pltpu.sync_copy(ptp_hbm, ptab_v)

        cbase = wid * c_per_w

        @pl.loop(0, n_cch)
        def _(ch):
            off = cbase + ch * c_chunk
            pltpu.sync_copy(ic_hbm.at[pl.ds(off, c_chunk)], idx_v)
            _vmem_gather_chunk(tab_v, idx_v, rb0, c_chunk, cl)
            pltpu.sync_copy(rb0, oc_hbm.at[pl.ds(off * cl, c_chunk * cl)])

        pbase = wid * p_per_w

        @pl.loop(0, n_pch)
        def _(ch):
            off = pbase + ch * p_chunk
            pltpu.sync_copy(ip_hbm.at[pl.ds(off, p_chunk)],
                            idx_v.at[pl.ds(0, p_chunk)])
            _vmem_gather_chunk(ptab_v, idx_v, rb1, p_chunk, plp)
            pltpu.sync_copy(rb1.at[pl.ds(0, p_chunk * plp)],
                            op_hbm.at[pl.ds(off * plp, p_chunk * plp)])

        def body(i_vmem, o_vmem):
            pltpu.sync_copy(wt_hbm.at[i_vmem.at[0]], o_vmem)

        pltpu.emit_pipeline(
            body,
            grid=(iw_hbm.shape[1] // _WINDOW,),
            in_specs=[pl.BlockSpec((1, _WINDOW), lambda i: (0, i))],
            out_specs=[pl.BlockSpec((_WINDOW, wd), lambda i: (i, 0))],
            core_axis_name=("c", "s"),
            dimension_semantics=(pltpu.PARALLEL,),
        )(iw_hbm, ow_hbm)

    word_g, char_g, pos_g = k(wt, ctp.reshape(-1), ptp.reshape(-1),
                              idx_w, idx_c, idx_p)
    return word_g, char_g.reshape(n_char, cl), pos_g.reshape(n_tok, plp)


def _tc_combine(char_g, word_g, pos_g, wcat, bias2d, t_blk, tp, to, kk, pd):
    bl, wd = word_g.shape
    plp = pos_g.shape[1]
    cl = char_g.shape[1]
    cd = wcat.shape[0] // kk
    f = wcat.shape[1]

    def _unpack(x):
        lo = lax.bitcast_convert_type(x << 16, jnp.float32)
        hi = lax.bitcast_convert_type(x & jnp.int32(-65536), jnp.float32)
        return jnp.concatenate([lo, hi], axis=1)

    def body(cg, wg, pg, wc, b, out):
        e = _unpack(cg[...]).astype(jnp.bfloat16)
        shifted = [e] + [
            jnp.concatenate([e[k:, :], e[:k, :]], axis=0) for k in range(1, kk)
        ]
        x = jnp.concatenate(shifted, axis=1)
        conv = lax.dot_general(
            x, wc[...], (((1,), (0,)), ((), ())),
            preferred_element_type=jnp.float32,
        )
        conv = conv + b[...]
        conv = conv.reshape(t_blk, tp, f)
        t_idx = lax.broadcasted_iota(jnp.int32, (t_blk, tp, f), 1)
        conv = jnp.where(t_idx < to, conv, -jnp.inf)
        feat = jnp.tanh(jnp.max(conv, axis=1))
        pos = _unpack(pg[...])
        out[:, 0:wd] = wg[...]
        out[:, wd:wd + f] = feat
        out[:, wd + f:] = pos
    return pl.pallas_call(
        body,
        grid=(bl // t_blk,),
        in_specs=[
            pl.BlockSpec((t_blk * tp, cl), lambda i: (i, 0)),
            pl.BlockSpec((t_blk, wd), lambda i: (i, 0)),
            pl.BlockSpec((t_blk, plp), lambda i: (i, 0)),
            pl.BlockSpec((kk * cd, f), lambda i: (0, 0)),
            pl.BlockSpec((1, f), lambda i: (0, 0)),
        ],
        out_specs=pl.BlockSpec((t_blk, wd + f + pd), lambda i: (i, 0)),
        out_shape=jax.ShapeDtypeStruct((bl, wd + f + pd), jnp.float32),
    )(char_g, word_g, pos_g, wcat, bias2d)


def kernel(input_word, input_char, input_pos, word_table, char_table, pos_table,
           conv_w, conv_b):
    b, l = input_word.shape
    lc = input_char.shape[2]
    f, cd, kk = conv_w.shape
    wd = word_table.shape[1]
    pd = pos_table.shape[1]
    bl = b * l
    tp = lc + 2 * (kk - 1)
    to = tp - kk + 1
    n_char_rows = char_table.shape[0]

    def _pack(tab):
        h = tab.shape[1] // 2
        pair = jnp.stack([tab[:, :h], tab[:, h:]], axis=-1)
        return lax.bitcast_convert_type(pair, jnp.int32)

    ct = jnp.concatenate(
        [char_table.astype(jnp.bfloat16),
         jnp.zeros((1, cd), jnp.bfloat16)], axis=0)
    ctp = _pack(ct)
    ptp = _pack(pos_table.astype(jnp.bfloat16))

    idx_c = jnp.pad(
        input_char.reshape(bl, lc), ((0, 0), (kk - 1, kk - 1)),
        constant_values=n_char_rows,
    ).reshape(bl * tp).astype(jnp.int32)
    idx_w = input_word.reshape(1, bl).astype(jnp.int32)
    idx_p = input_pos.reshape(bl).astype(jnp.int32)

    word_g, char_g, pos_g = _sc_gather_all(
        idx_w, idx_c, idx_p, word_table, ctp, ptp)

    wcat = jnp.transpose(conv_w, (2, 1, 0)).reshape(kk * cd, f)
    wcat = wcat.astype(jnp.bfloat16)

    out = _tc_combine(char_g, word_g, pos_g, wcat, conv_b.reshape(1, f),
                      256, tp, to, kk, pd)
    return out.reshape(b, l, wd + f + pd)


# speedup vs baseline: 4.5871x; 1.2614x over previous
import dataclasses
import functools

import jax
import jax.numpy as jnp
from jax import lax
from jax.experimental import pallas as pl
from jax.experimental.pallas import tpu as pltpu
from jax.experimental.pallas import tpu_sc as plsc

_WINDOW = 128
_NC, _NS = 2, 16
_NW = _NC * _NS


def _vmem_gather_chunk(tab_v, idx_v, idx_off, row_buf, n_rows, lanes):
    @pl.loop(0, n_rows // 16)
    def _(j):
        iv = idx_v[pl.ds(idx_off + j * 16, 16)]
        src_base = iv * lanes
        dst_base = (j * 16 + lax.iota(jnp.int32, 16)) * lanes
        for c0 in range(0, lanes, 8):
            vs = [plsc.load_gather(tab_v, [src_base + (c0 + u)])
                  for u in range(8)]
            for u in range(8):
                plsc.store_scatter(row_buf, [dst_base + (c0 + u)], vs[u])


def _sc_gather_all(idx_w, idx_c, idx_p, wt, ctp, ptp):
    wd = wt.shape[1]
    ncr, cl = ctp.shape
    npr, plp = ptp.shape
    n_tok = idx_w.shape[1]
    n_char = idx_c.shape[0]
    c_per_w = n_char // _NW
    p_per_w = n_tok // _NW
    c_chunk = 600
    p_chunk = 400
    n_cch = c_per_w // c_chunk
    n_pch = p_per_w // p_chunk
    mesh = plsc.VectorSubcoreMesh(core_axis_name="c", subcore_axis_name="s")
    out_types = (
        jax.ShapeDtypeStruct((n_tok, wd), wt.dtype),
        jax.ShapeDtypeStruct((n_char * cl,), jnp.int32),
        jax.ShapeDtypeStruct((n_tok * plp,), jnp.int32),
    )
    scratch = [
        pltpu.VMEM((ncr * cl,), jnp.int32),
        pltpu.VMEM((npr * plp,), jnp.int32),
        pltpu.VMEM((2 * c_chunk,), jnp.int32),
        pltpu.VMEM((c_chunk * cl,), jnp.int32),
        pltpu.VMEM((c_chunk * cl,), jnp.int32),
        pltpu.SemaphoreType.DMA,
        pltpu.SemaphoreType.DMA,
    ]

    cp = pltpu.CompilerParams()
    if "needs_layout_passes" in pltpu.CompilerParams.__dataclass_fields__:
        cp = dataclasses.replace(cp, needs_layout_passes=False)

    @functools.partial(pl.kernel, out_type=out_types, mesh=mesh,
                       scratch_types=scratch, compiler_params=cp)
    def k(wt_hbm, ctp_hbm, ptp_hbm, iw_hbm, ic_hbm, ip_hbm,
          ow_hbm, oc_hbm, op_hbm, tab_v, ptab_v, idx_v, rb0, rb1, s0, s1):
        wid = lax.axis_index("s") * _NC + lax.axis_index("c")
        pltpu.sync_copy(ctp_hbm, tab_v)
        pltpu.sync_copy(ptp_hbm, ptab_v)

        cbase = wid * c_per_w
        cbytes = c_chunk * cl

        def _wait_out(buf, sem):
            pltpu.make_async_copy(buf, oc_hbm.at[pl.ds(0, cbytes)], sem).wait()

        @pl.loop(0, n_cch, step=2)
        def _(ch):
            off = cbase + ch * c_chunk
            pltpu.sync_copy(ic_hbm.at[pl.ds(off, 2 * c_chunk)], idx_v)

            @pl.when(ch > 0)
            def _():
                _wait_out(rb0, s0)

            _vmem_gather_chunk(tab_v, idx_v, 0, rb0, c_chunk, cl)
            pltpu.async_copy(rb0, oc_hbm.at[pl.ds(off * cl, cbytes)], s0)

            @pl.when(ch > 0)
            def _():
                _wait_out(rb1, s1)

            _vmem_gather_chunk(tab_v, idx_v, c_chunk, rb1, c_chunk, cl)
            pltpu.async_copy(
                rb1, oc_hbm.at[pl.ds((off + c_chunk) * cl, cbytes)], s1)

        _wait_out(rb0, s0)
        _wait_out(rb1, s1)

        pbase = wid * p_per_w

        @pl.loop(0, n_pch)
        def _(ch):
            off = pbase + ch * p_chunk
            pltpu.sync_copy(ip_hbm.at[pl.ds(off, p_chunk)],
                            idx_v.at[pl.ds(0, p_chunk)])
            _vmem_gather_chunk(ptab_v, idx_v, 0, rb1, p_chunk, plp)
            pltpu.sync_copy(rb1.at[pl.ds(0, p_chunk * plp)],
                            op_hbm.at[pl.ds(off * plp, p_chunk * plp)])

        def body(i_vmem, o_vmem):
            pltpu.sync_copy(wt_hbm.at[i_vmem.at[0]], o_vmem)

        pltpu.emit_pipeline(
            body,
            grid=(iw_hbm.shape[1] // _WINDOW,),
            in_specs=[pl.BlockSpec((1, _WINDOW), lambda i: (0, i))],
            out_specs=[pl.BlockSpec((_WINDOW, wd), lambda i: (i, 0))],
            core_axis_name=("c", "s"),
            dimension_semantics=(pltpu.PARALLEL,),
        )(iw_hbm, ow_hbm)

    word_g, char_g, pos_g = k(wt, ctp.reshape(-1), ptp.reshape(-1),
                              idx_w, idx_c, idx_p)
    return word_g, char_g.reshape(n_char, cl), pos_g.reshape(n_tok, plp)


def _tc_combine(char_g, word_g, pos_g, wcat, bias2d, t_blk, tp, to, kk, pd):
    bl, wd = word_g.shape
    plp = pos_g.shape[1]
    cl = char_g.shape[1]
    cd = wcat.shape[0] // kk
    f = wcat.shape[1]

    def _unpack(x):
        lo = lax.bitcast_convert_type(x << 16, jnp.float32)
        hi = lax.bitcast_convert_type(x & jnp.int32(-65536), jnp.float32)
        return jnp.concatenate([lo, hi], axis=1)

    def body(cg, wg, pg, wc, b, out):
        e = _unpack(cg[...]).astype(jnp.bfloat16)
        shifted = [e] + [
            jnp.concatenate([e[k:, :], e[:k, :]], axis=0) for k in range(1, kk)
        ]
        x = jnp.concatenate(shifted, axis=1)
        conv = lax.dot_general(
            x, wc[...], (((1,), (0,)), ((), ())),
            preferred_element_type=jnp.float32,
        )
        conv = conv + b[...]
        conv = conv.reshape(t_blk, tp, f)
        t_idx = lax.broadcasted_iota(jnp.int32, (t_blk, tp, f), 1)
        conv = jnp.where(t_idx < to, conv, -jnp.inf)
        feat = jnp.tanh(jnp.max(conv, axis=1))
        pos = _unpack(pg[...])
        out[:, 0:wd] = wg[...]
        out[:, wd:wd + f] = feat
        out[:, wd + f:] = pos
    return pl.pallas_call(
        body,
        grid=(bl // t_blk,),
        in_specs=[
            pl.BlockSpec((t_blk * tp, cl), lambda i: (i, 0)),
            pl.BlockSpec((t_blk, wd), lambda i: (i, 0)),
            pl.BlockSpec((t_blk, plp), lambda i: (i, 0)),
            pl.BlockSpec((kk * cd, f), lambda i: (0, 0)),
            pl.BlockSpec((1, f), lambda i: (0, 0)),
        ],
        out_specs=pl.BlockSpec((t_blk, wd + f + pd), lambda i: (i, 0)),
        out_shape=jax.ShapeDtypeStruct((bl, wd + f + pd), jnp.float32),
    )(char_g, word_g, pos_g, wcat, bias2d)


def kernel(input_word, input_char, input_pos, word_table, char_table, pos_table,
           conv_w, conv_b):
    b, l = input_word.shape
    lc = input_char.shape[2]
    f, cd, kk = conv_w.shape
    wd = word_table.shape[1]
    pd = pos_table.shape[1]
    bl = b * l
    tp = lc + 2 * (kk - 1)
    to = tp - kk + 1
    n_char_rows = char_table.shape[0]

    def _pack(tab):
        h = tab.shape[1] // 2
        pair = jnp.stack([tab[:, :h], tab[:, h:]], axis=-1)
        return lax.bitcast_convert_type(pair, jnp.int32)

    ct = jnp.concatenate(
        [char_table.astype(jnp.bfloat16),
         jnp.zeros((1, cd), jnp.bfloat16)], axis=0)
    ctp = _pack(ct)
    ptp = _pack(pos_table.astype(jnp.bfloat16))

    idx_c = jnp.pad(
        input_char.reshape(bl, lc), ((0, 0), (kk - 1, kk - 1)),
        constant_values=n_char_rows,
    ).reshape(bl * tp).astype(jnp.int32)
    idx_w = input_word.reshape(1, bl).astype(jnp.int32)
    idx_p = input_pos.reshape(bl).astype(jnp.int32)

    word_g, char_g, pos_g = _sc_gather_all(
        idx_w, idx_c, idx_p, word_table, ctp, ptp)

    wcat = jnp.transpose(conv_w, (2, 1, 0)).reshape(kk * cd, f)
    wcat = wcat.astype(jnp.bfloat16)

    out = _tc_combine(char_g, word_g, pos_g, wcat, conv_b.reshape(1, f),
                      256, tp, to, kk, pd)
    return out.reshape(b, l, wd + f + pd)


# speedup vs baseline: 7.6919x; 1.6768x over previous
import dataclasses
import functools

import jax
import jax.numpy as jnp
from jax import lax
from jax.experimental import pallas as pl
from jax.experimental.pallas import tpu as pltpu
from jax.experimental.pallas import tpu_sc as plsc

_WINDOW = 128
_NC, _NS = 2, 16
_NW = _NC * _NS


def _vmem_gather_chunk(tab_v, idx_v, idx_off, row_buf, n_rows, lanes):
    nw = lanes // 16

    @pl.loop(0, n_rows, step=16)
    def _(r):
        iv = idx_v[pl.ds(idx_off + r, 16)]
        srcs = [iv[u] * lanes for u in range(16)]
        vals = [tab_v[pl.ds(s + w * 16, 16)] for s in srcs for w in range(nw)]
        for u in range(16):
            for w in range(nw):
                row_buf[pl.ds((r + u) * lanes + w * 16, 16)] = (
                    vals[u * nw + w])


def _sc_gather_all(idx_w, idx_c, idx_p, wt, ctp, ptp):
    wd = wt.shape[1]
    ncr, cl = ctp.shape
    npr, plp = ptp.shape
    n_tok = idx_w.shape[1]
    n_char = idx_c.shape[0]
    c_per_w = n_char // _NW
    p_per_w = n_tok // _NW
    c_chunk = 640
    p_chunk = 400
    n_cch = c_per_w // c_chunk
    n_pch = p_per_w // p_chunk
    mesh = plsc.VectorSubcoreMesh(core_axis_name="c", subcore_axis_name="s")
    out_types = (
        jax.ShapeDtypeStruct((n_tok, wd), wt.dtype),
        jax.ShapeDtypeStruct((n_char * cl,), jnp.int32),
        jax.ShapeDtypeStruct((n_tok * plp,), jnp.int32),
    )
    scratch = [
        pltpu.VMEM((ncr * cl,), jnp.int32),
        pltpu.VMEM((npr * plp,), jnp.int32),
        pltpu.VMEM((2 * c_chunk,), jnp.int32),
        pltpu.VMEM((c_chunk * cl,), jnp.int32),
        pltpu.VMEM((c_chunk * cl,), jnp.int32),
        pltpu.SemaphoreType.DMA,
        pltpu.SemaphoreType.DMA,
    ]

    cp = pltpu.CompilerParams()
    if "needs_layout_passes" in pltpu.CompilerParams.__dataclass_fields__:
        cp = dataclasses.replace(cp, needs_layout_passes=False)

    @functools.partial(pl.kernel, out_type=out_types, mesh=mesh,
                       scratch_types=scratch, compiler_params=cp)
    def k(wt_hbm, ctp_hbm, ptp_hbm, iw_hbm, ic_hbm, ip_hbm,
          ow_hbm, oc_hbm, op_hbm, tab_v, ptab_v, idx_v, rb0, rb1, s0, s1):
        wid = lax.axis_index("s") * _NC + lax.axis_index("c")
        pltpu.sync_copy(ctp_hbm, tab_v)
        pltpu.sync_copy(ptp_hbm, ptab_v)

        cbase = wid * c_per_w
        cbytes = c_chunk * cl

        def _wait_out(buf, sem):
            pltpu.make_async_copy(buf, oc_hbm.at[pl.ds(0, cbytes)], sem).wait()

        @pl.loop(0, n_cch, step=2)
        def _(ch):
            off = cbase + ch * c_chunk
            pltpu.sync_copy(ic_hbm.at[pl.ds(off, 2 * c_chunk)], idx_v)

            @pl.when(ch > 0)
            def _():
                _wait_out(rb0, s0)

            _vmem_gather_chunk(tab_v, idx_v, 0, rb0, c_chunk, cl)
            pltpu.async_copy(rb0, oc_hbm.at[pl.ds(off * cl, cbytes)], s0)

            @pl.when(ch > 0)
            def _():
                _wait_out(rb1, s1)

            _vmem_gather_chunk(tab_v, idx_v, c_chunk, rb1, c_chunk, cl)
            pltpu.async_copy(
                rb1, oc_hbm.at[pl.ds((off + c_chunk) * cl, cbytes)], s1)

        _wait_out(rb0, s0)
        _wait_out(rb1, s1)

        pbase = wid * p_per_w

        @pl.loop(0, n_pch)
        def _(ch):
            off = pbase + ch * p_chunk
            pltpu.sync_copy(ip_hbm.at[pl.ds(off, p_chunk)],
                            idx_v.at[pl.ds(0, p_chunk)])
            _vmem_gather_chunk(ptab_v, idx_v, 0, rb1, p_chunk, plp)
            pltpu.sync_copy(rb1.at[pl.ds(0, p_chunk * plp)],
                            op_hbm.at[pl.ds(off * plp, p_chunk * plp)])

        def body(i_vmem, o_vmem):
            pltpu.sync_copy(wt_hbm.at[i_vmem.at[0]], o_vmem)

        pltpu.emit_pipeline(
            body,
            grid=(iw_hbm.shape[1] // _WINDOW,),
            in_specs=[pl.BlockSpec((1, _WINDOW), lambda i: (0, i))],
            out_specs=[pl.BlockSpec((_WINDOW, wd), lambda i: (i, 0))],
            core_axis_name=("c", "s"),
            dimension_semantics=(pltpu.PARALLEL,),
        )(iw_hbm, ow_hbm)

    word_g, char_g, pos_g = k(wt, ctp.reshape(-1), ptp.reshape(-1),
                              idx_w, idx_c, idx_p)
    return word_g, char_g.reshape(n_char, cl), pos_g.reshape(n_tok, plp)


def _tc_combine(char_g, word_g, pos_g, wcat, bias2d, t_blk, tp, to, kk, pd):
    bl, wd = word_g.shape
    plp = pos_g.shape[1]
    cl = char_g.shape[1]
    cd = wcat.shape[0] // kk
    f = wcat.shape[1]

    def _unpack(x):
        lo = lax.bitcast_convert_type(x << 16, jnp.float32)
        hi = lax.bitcast_convert_type(x & jnp.int32(-65536), jnp.float32)
        return jnp.concatenate([lo, hi], axis=1)

    def body(cg, wg, pg, wc, b, out):
        e = _unpack(cg[...]).astype(jnp.bfloat16)
        shifted = [e] + [
            jnp.concatenate([e[k:, :], e[:k, :]], axis=0) for k in range(1, kk)
        ]
        x = jnp.concatenate(shifted, axis=1)
        conv = lax.dot_general(
            x, wc[...], (((1,), (0,)), ((), ())),
            preferred_element_type=jnp.float32,
        )
        conv = conv + b[...]
        conv = conv.reshape(t_blk, tp, f)
        t_idx = lax.broadcasted_iota(jnp.int32, (t_blk, tp, f), 1)
        conv = jnp.where(t_idx < to, conv, -jnp.inf)
        feat = jnp.tanh(jnp.max(conv, axis=1))
        pos = _unpack(pg[...])
        out[:, 0:wd] = wg[...]
        out[:, wd:wd + f] = feat
        out[:, wd + f:] = pos
    return pl.pallas_call(
        body,
        grid=(bl // t_blk,),
        in_specs=[
            pl.BlockSpec((t_blk * tp, cl), lambda i: (i, 0)),
            pl.BlockSpec((t_blk, wd), lambda i: (i, 0)),
            pl.BlockSpec((t_blk, plp), lambda i: (i, 0)),
            pl.BlockSpec((kk * cd, f), lambda i: (0, 0)),
            pl.BlockSpec((1, f), lambda i: (0, 0)),
        ],
        out_specs=pl.BlockSpec((t_blk, wd + f + pd), lambda i: (i, 0)),
        out_shape=jax.ShapeDtypeStruct((bl, wd + f + pd), jnp.float32),
    )(char_g, word_g, pos_g, wcat, bias2d)


def kernel(input_word, input_char, input_pos, word_table, char_table, pos_table,
           conv_w, conv_b):
    b, l = input_word.shape
    lc = input_char.shape[2]
    f, cd, kk = conv_w.shape
    wd = word_table.shape[1]
    pd = pos_table.shape[1]
    bl = b * l
    tp = lc + 2 * (kk - 1)
    to = tp - kk + 1
    n_char_rows = char_table.shape[0]

    def _pack(tab):
        h = tab.shape[1] // 2
        pair = jnp.stack([tab[:, :h], tab[:, h:]], axis=-1)
        return lax.bitcast_convert_type(pair, jnp.int32)

    ct = jnp.concatenate(
        [char_table.astype(jnp.bfloat16),
         jnp.zeros((1, cd), jnp.bfloat16)], axis=0)
    ctp = _pack(ct)
    ptp = _pack(pos_table.astype(jnp.bfloat16))

    idx_c = jnp.pad(
        input_char.reshape(bl, lc), ((0, 0), (kk - 1, kk - 1)),
        constant_values=n_char_rows,
    ).reshape(bl * tp).astype(jnp.int32)
    idx_w = input_word.reshape(1, bl).astype(jnp.int32)
    idx_p = input_pos.reshape(bl).astype(jnp.int32)

    word_g, char_g, pos_g = _sc_gather_all(
        idx_w, idx_c, idx_p, word_table, ctp, ptp)

    wcat = jnp.transpose(conv_w, (2, 1, 0)).reshape(kk * cd, f)
    wcat = wcat.astype(jnp.bfloat16)

    out = _tc_combine(char_g, word_g, pos_g, wcat, conv_b.reshape(1, f),
                      256, tp, to, kk, pd)
    return out.reshape(b, l, wd + f + pd)


# speedup vs baseline: 7.8134x; 1.0158x over previous
import dataclasses
import functools

import jax
import jax.numpy as jnp
from jax import lax
from jax.experimental import pallas as pl
from jax.experimental.pallas import tpu as pltpu
from jax.experimental.pallas import tpu_sc as plsc

_WINDOW = 128
_NC, _NS = 2, 16
_NW = _NC * _NS


def _vmem_gather_chunk(tab_v, idx_v, idx_off, row_buf, n_rows, lanes):
    nw = lanes // 16

    @pl.loop(0, n_rows, step=16)
    def _(r):
        iv = idx_v[pl.ds(idx_off + r, 16)]
        srcs = [iv[u] * lanes for u in range(16)]
        vals = [tab_v[pl.ds(s + w * 16, 16)] for s in srcs for w in range(nw)]
        for u in range(16):
            for w in range(nw):
                row_buf[pl.ds((r + u) * lanes + w * 16, 16)] = (
                    vals[u * nw + w])


def _sc_gather_all(idx_w, idx_c, idx_p, wt, ctp, ptp):
    wd = wt.shape[1]
    ncr, cl = ctp.shape
    npr, plp = ptp.shape
    n_tok = idx_w.shape[1]
    n_char = idx_c.shape[0]
    c_per_w = n_char // _NW
    p_per_w = n_tok // _NW
    c_chunk = 640
    p_chunk = 400
    n_cch = c_per_w // c_chunk
    n_pch = p_per_w // p_chunk
    mesh = plsc.VectorSubcoreMesh(core_axis_name="c", subcore_axis_name="s")
    out_types = (
        jax.ShapeDtypeStruct((n_tok, wd), wt.dtype),
        jax.ShapeDtypeStruct((n_char * cl,), jnp.int32),
        jax.ShapeDtypeStruct((n_tok * plp,), jnp.int32),
    )
    scratch = [
        pltpu.VMEM((ncr * cl,), jnp.int32),
        pltpu.VMEM((npr * plp,), jnp.int32),
        pltpu.VMEM((2 * c_chunk,), jnp.int32),
        pltpu.VMEM((c_chunk * cl,), jnp.int32),
        pltpu.VMEM((c_chunk * cl,), jnp.int32),
        pltpu.SemaphoreType.DMA,
        pltpu.SemaphoreType.DMA,
    ]

    cp = pltpu.CompilerParams()
    if "needs_layout_passes" in pltpu.CompilerParams.__dataclass_fields__:
        cp = dataclasses.replace(cp, needs_layout_passes=False)

    @functools.partial(pl.kernel, out_type=out_types, mesh=mesh,
                       scratch_types=scratch, compiler_params=cp)
    def k(wt_hbm, ctp_hbm, ptp_hbm, iw_hbm, ic_hbm, ip_hbm,
          ow_hbm, oc_hbm, op_hbm, tab_v, ptab_v, idx_v, rb0, rb1, s0, s1):
        wid = lax.axis_index("s") * _NC + lax.axis_index("c")
        pltpu.sync_copy(ctp_hbm, tab_v)
        pltpu.sync_copy(ptp_hbm, ptab_v)

        cbase = wid * c_per_w
        cbytes = c_chunk * cl

        def _wait_out(buf, sem):
            pltpu.make_async_copy(buf, oc_hbm.at[pl.ds(0, cbytes)], sem).wait()

        @pl.loop(0, n_cch, step=2)
        def _(ch):
            off = cbase + ch * c_chunk
            pltpu.sync_copy(ic_hbm.at[pl.ds(off, 2 * c_chunk)], idx_v)

            @pl.when(ch > 0)
            def _():
                _wait_out(rb0, s0)

            _vmem_gather_chunk(tab_v, idx_v, 0, rb0, c_chunk, cl)
            pltpu.async_copy(rb0, oc_hbm.at[pl.ds(off * cl, cbytes)], s0)

            @pl.when(ch > 0)
            def _():
                _wait_out(rb1, s1)

            _vmem_gather_chunk(tab_v, idx_v, c_chunk, rb1, c_chunk, cl)
            pltpu.async_copy(
                rb1, oc_hbm.at[pl.ds((off + c_chunk) * cl, cbytes)], s1)

        _wait_out(rb0, s0)
        _wait_out(rb1, s1)

        pbase = wid * p_per_w

        @pl.loop(0, n_pch)
        def _(ch):
            off = pbase + ch * p_chunk
            pltpu.sync_copy(ip_hbm.at[pl.ds(off, p_chunk)],
                            idx_v.at[pl.ds(0, p_chunk)])
            _vmem_gather_chunk(ptab_v, idx_v, 0, rb1, p_chunk, plp)
            pltpu.sync_copy(rb1.at[pl.ds(0, p_chunk * plp)],
                            op_hbm.at[pl.ds(off * plp, p_chunk * plp)])

        def body(i_vmem, o_vmem):
            pltpu.sync_copy(wt_hbm.at[i_vmem.at[0]], o_vmem)

        pltpu.emit_pipeline(
            body,
            grid=(iw_hbm.shape[1] // _WINDOW,),
            in_specs=[pl.BlockSpec((1, _WINDOW), lambda i: (0, i))],
            out_specs=[pl.BlockSpec((_WINDOW, wd), lambda i: (i, 0))],
            core_axis_name=("c", "s"),
            dimension_semantics=(pltpu.PARALLEL,),
        )(iw_hbm, ow_hbm)

    word_g, char_g, pos_g = k(wt, ctp.reshape(-1), ptp.reshape(-1),
                              idx_w, idx_c, idx_p)
    return word_g, char_g.reshape(n_char, cl), pos_g.reshape(n_tok, plp)


def _tc_combine(char_g, word_g, pos_g, wcat, bias2d, t_blk, tp, to, kk, pd):
    bl, wd = word_g.shape
    plp = pos_g.shape[1]
    cl = char_g.shape[1]
    cd = wcat.shape[0] // kk
    f = wcat.shape[1]

    def _unpack(x):
        lo = lax.bitcast_convert_type(x << 16, jnp.float32)
        hi = lax.bitcast_convert_type(x & jnp.int32(-65536), jnp.float32)
        return jnp.concatenate([lo, hi], axis=1)

    def body(cg, wg, pg, wc, b, out):
        e = _unpack(cg[...]).astype(jnp.bfloat16)
        shifted = [e] + [
            jnp.concatenate([e[k:, :], e[:k, :]], axis=0) for k in range(1, kk)
        ]
        x = jnp.concatenate(shifted, axis=1)
        conv = lax.dot_general(
            x, wc[...], (((1,), (0,)), ((), ())),
            preferred_element_type=jnp.float32,
        )
        conv = conv + b[...]
        conv = conv.reshape(t_blk, tp, f)
        t_idx = lax.broadcasted_iota(jnp.int32, (t_blk, tp, f), 1)
        conv = jnp.where(t_idx < to, conv, -jnp.inf)
        feat = jnp.tanh(jnp.max(conv, axis=1))
        pos = _unpack(pg[...])
        out[:, 0:wd] = wg[...]
        out[:, wd:wd + f] = feat
        out[:, wd + f:] = pos
    return pl.pallas_call(
        body,
        grid=(bl // t_blk,),
        in_specs=[
            pl.BlockSpec((t_blk * tp, cl), lambda i: (i, 0)),
            pl.BlockSpec((t_blk, wd), lambda i: (i, 0)),
            pl.BlockSpec((t_blk, plp), lambda i: (i, 0)),
            pl.BlockSpec((kk * cd, f), lambda i: (0, 0)),
            pl.BlockSpec((1, f), lambda i: (0, 0)),
        ],
        out_specs=pl.BlockSpec((t_blk, wd + f + pd), lambda i: (i, 0)),
        out_shape=jax.ShapeDtypeStruct((bl, wd + f + pd), jnp.float32),
    )(char_g, word_g, pos_g, wcat, bias2d)


def kernel(input_word, input_char, input_pos, word_table, char_table, pos_table,
           conv_w, conv_b):
    b, l = input_word.shape
    lc = input_char.shape[2]
    f, cd, kk = conv_w.shape
    wd = word_table.shape[1]
    pd = pos_table.shape[1]
    bl = b * l
    tp = lc + 2 * (kk - 1)
    to = tp - kk + 1
    n_char_rows = char_table.shape[0]

    def _pack(tab):
        h = tab.shape[1] // 2
        pair = jnp.stack([tab[:, :h], tab[:, h:]], axis=-1)
        return lax.bitcast_convert_type(pair, jnp.int32)

    ct = jnp.concatenate(
        [char_table.astype(jnp.bfloat16),
         jnp.zeros((1, cd), jnp.bfloat16)], axis=0)
    ctp = _pack(ct)
    ptp = _pack(pos_table.astype(jnp.bfloat16))

    idx_c = jnp.pad(
        input_char.reshape(bl, lc), ((0, 0), (kk - 1, kk - 1)),
        constant_values=n_char_rows,
    ).reshape(bl * tp).astype(jnp.int32)
    idx_w = input_word.reshape(1, bl).astype(jnp.int32)
    idx_p = input_pos.reshape(bl).astype(jnp.int32)

    word_g, char_g, pos_g = _sc_gather_all(
        idx_w, idx_c, idx_p, word_table, ctp, ptp)

    wcat = jnp.transpose(conv_w, (2, 1, 0)).reshape(kk * cd, f)
    wcat = wcat.astype(jnp.bfloat16)

    out = _tc_combine(char_g, word_g, pos_g, wcat, conv_b.reshape(1, f),
                      512, tp, to, kk, pd)
    return out.reshape(b, l, wd + f + pd)
